# baseline (speedup 1.0000x reference)
import functools

import jax
import jax.numpy as jnp
from jax import lax
from jax.experimental import pallas as pl
from jax.experimental.pallas import tpu as pltpu
from jax.experimental.pallas import tpu_sc as plsc


@functools.cache
def _make_gather(VP, DP, B):
    info = plsc.get_sparse_core_info()
    NC, NS = info.num_cores, info.num_subcores
    NW = NC * NS
    assert B % NW == 0
    b_per_w = B // NW
    CHUNK = 400
    assert b_per_w % (2 * CHUNK) == 0
    n_pairs = b_per_w // (2 * CHUNK)
    mesh = plsc.VectorSubcoreMesh(core_axis_name="c", subcore_axis_name="s")

    @functools.partial(
        pl.kernel,
        mesh=mesh,
        out_type=jax.ShapeDtypeStruct((B, DP), jnp.float32),
        scratch_types=[
            pltpu.VMEM((b_per_w,), jnp.int32),
            pltpu.VMEM((CHUNK, DP), jnp.float32),
            pltpu.VMEM((CHUNK, DP), jnp.float32),
            pltpu.SemaphoreType.DMA,
            pltpu.SemaphoreType.DMA,
            pltpu.SemaphoreType.DMA,
            pltpu.SemaphoreType.DMA,
        ],
    )
    def gather_kernel(idx_hbm, table_hbm, out_hbm, idx_all, rows0, rows1,
                      gsem0, gsem1, osem0, osem1):
        wid = lax.axis_index("s") * NC + lax.axis_index("c")
        base = wid * b_per_w
        pltpu.sync_copy(idx_hbm.at[pl.ds(base, b_per_w)], idx_all)

        def g_start(c, buf, sem):
            return pltpu.async_copy(
                table_hbm.at[idx_all.at[pl.ds(c * CHUNK, CHUNK)]], buf, sem)

        def s_start(c, buf, sem):
            return pltpu.async_copy(
                buf, out_hbm.at[pl.ds(base + c * CHUNK, CHUNK)], sem)

        def g_wait(buf, sem):
            pltpu.make_async_copy(table_hbm.at[idx_all.at[pl.ds(0, CHUNK)]],
                                  buf, sem).wait()

        def s_wait(buf, sem):
            pltpu.make_async_copy(buf, out_hbm.at[pl.ds(base, CHUNK)],
                                  sem).wait()

        g_start(0, rows0, gsem0)

        def body(p, carry):
            c0 = 2 * p
            c1 = c0 + 1
            g_wait(rows0, gsem0)
            s_start(c0, rows0, osem0)

            @pl.when(p > 0)
            def _():
                s_wait(rows1, osem1)

            g_start(c1, rows1, gsem1)
            g_wait(rows1, gsem1)
            s_start(c1, rows1, osem1)

            @pl.when(p < n_pairs - 1)
            def _():
                s_wait(rows0, osem0)
                g_start(c0 + 2, rows0, gsem0)

            return carry

        lax.fori_loop(0, n_pairs, body, 0)
        s_wait(rows0, osem0)
        s_wait(rows1, osem1)

    return gather_kernel


@jax.jit
def kernel(x, weight):
    B, H = x.shape
    V, D = weight.shape
    table2 = weight.reshape(V // 2, 2 * D)
    pair_idx = (x >> 1).reshape(B * H)
    pairs = _make_gather(V // 2, 2 * D, B * H)(pair_idx, table2)
    pairs3 = pairs.reshape(B, H, 2 * D)
    odd = (x & 1).astype(bool)[..., None]
    return jnp.where(odd, pairs3[..., D:], pairs3[..., :D])


# speedup vs baseline: 1.6984x; 1.6984x over previous
import functools

import jax
import jax.numpy as jnp
from jax import lax
from jax.experimental import pallas as pl
from jax.experimental.pallas import tpu as pltpu
from jax.experimental.pallas import tpu_sc as plsc


@functools.cache
def _make_gather(V, D, BATCH, HIST):
    B = BATCH * HIST
    info = plsc.get_sparse_core_info()
    NC, NS = info.num_cores, info.num_subcores
    NW = NC * NS
    assert B % NW == 0 and BATCH % NW == 0
    b_per_w = B // NW
    rows_per_w = BATCH // NW
    CHUNK = HIST
    assert rows_per_w % 2 == 0
    n_pairs = rows_per_w // 2
    mesh = plsc.VectorSubcoreMesh(core_axis_name="c", subcore_axis_name="s")

    @functools.partial(
        pl.kernel,
        mesh=mesh,
        out_type=jax.ShapeDtypeStruct((BATCH, HIST, D), jnp.float32),
        scratch_types=[
            pltpu.VMEM((b_per_w,), jnp.int32),
            pltpu.VMEM((CHUNK, D), jnp.float32),
            pltpu.VMEM((CHUNK, D), jnp.float32),
            pltpu.SemaphoreType.DMA,
            pltpu.SemaphoreType.DMA,
            pltpu.SemaphoreType.DMA,
            pltpu.SemaphoreType.DMA,
        ],
        compiler_params=pltpu.CompilerParams(use_tc_tiling_on_sc=False),
    )
    def gather_kernel(idx_hbm, table_hbm, out3_hbm, idx_all, rows0, rows1,
                      gsem0, gsem1, osem0, osem1):
        wid = lax.axis_index("s") * NC + lax.axis_index("c")
        base = wid * b_per_w
        brow = wid * rows_per_w
        pltpu.sync_copy(idx_hbm.at[pl.ds(base, b_per_w)], idx_all)

        def g_start(c, buf, sem):
            return pltpu.async_copy(
                table_hbm.at[idx_all.at[pl.ds(c * CHUNK, CHUNK)]], buf, sem)

        def s_start(c, buf, sem):
            return pltpu.async_copy(buf, out3_hbm.at[brow + c], sem)

        def g_wait(buf, sem):
            pltpu.make_async_copy(table_hbm.at[idx_all.at[pl.ds(0, CHUNK)]],
                                  buf, sem).wait()

        def s_wait(buf, sem):
            pltpu.make_async_copy(buf, out3_hbm.at[brow], sem).wait()

        g_start(0, rows0, gsem0)

        def body(p, carry):
            c0 = 2 * p
            c1 = c0 + 1
            g_wait(rows0, gsem0)
            s_start(c0, rows0, osem0)

            @pl.when(p > 0)
            def _():
                s_wait(rows1, osem1)

            g_start(c1, rows1, gsem1)
            g_wait(rows1, gsem1)
            s_start(c1, rows1, osem1)

            @pl.when(p < n_pairs - 1)
            def _():
                s_wait(rows0, osem0)
                g_start(c0 + 2, rows0, gsem0)

            return carry

        lax.fori_loop(0, n_pairs, body, 0)
        s_wait(rows0, osem0)
        s_wait(rows1, osem1)

    return gather_kernel


@jax.jit
def kernel(x, weight):
    B, H = x.shape
    V, D = weight.shape
    flat = x.reshape(B * H)
    return _make_gather(V, D, B, H)(flat, weight)


# speedup vs baseline: 1.7572x; 1.0347x over previous
import functools

import jax
import jax.numpy as jnp
from jax import lax
from jax.experimental import pallas as pl
from jax.experimental.pallas import tpu as pltpu
from jax.experimental.pallas import tpu_sc as plsc


@functools.cache
def _make_gather(V, D, B):
    info = plsc.get_sparse_core_info()
    NC, NS = info.num_cores, info.num_subcores
    NW = NC * NS
    assert B % NW == 0
    b_per_w = B // NW
    CHUNK = 800
    assert b_per_w % (2 * CHUNK) == 0
    n_pairs = b_per_w // (2 * CHUNK)
    mesh = plsc.VectorSubcoreMesh(core_axis_name="c", subcore_axis_name="s")

    @functools.partial(
        pl.kernel,
        mesh=mesh,
        out_type=jax.ShapeDtypeStruct((B, D), jnp.float32),
        scratch_types=[
            pltpu.VMEM((b_per_w,), jnp.int32),
            pltpu.VMEM((CHUNK, D), jnp.float32),
            pltpu.VMEM((CHUNK, D), jnp.float32),
            pltpu.SemaphoreType.DMA,
            pltpu.SemaphoreType.DMA,
            pltpu.SemaphoreType.DMA,
            pltpu.SemaphoreType.DMA,
        ],
        compiler_params=pltpu.CompilerParams(use_tc_tiling_on_sc=False),
    )
    def gather_kernel(idx_hbm, table_hbm, out_hbm, idx_all, rows0, rows1,
                      gsem0, gsem1, osem0, osem1):
        wid = lax.axis_index("s") * NC + lax.axis_index("c")
        base = wid * b_per_w
        pltpu.sync_copy(idx_hbm.at[pl.ds(base, b_per_w)], idx_all)

        def g_start(c, buf, sem):
            return pltpu.async_copy(
                table_hbm.at[idx_all.at[pl.ds(c * CHUNK, CHUNK)]], buf, sem)

        def s_start(c, buf, sem):
            return pltpu.async_copy(
                buf, out_hbm.at[pl.ds(base + c * CHUNK, CHUNK)], sem)

        def g_wait(buf, sem):
            pltpu.make_async_copy(table_hbm.at[idx_all.at[pl.ds(0, CHUNK)]],
                                  buf, sem).wait()

        def s_wait(buf, sem):
            pltpu.make_async_copy(buf, out_hbm.at[pl.ds(base, CHUNK)],
                                  sem).wait()

        g_start(0, rows0, gsem0)

        def body(p, carry):
            c0 = 2 * p
            c1 = c0 + 1
            g_wait(rows0, gsem0)
            s_start(c0, rows0, osem0)

            @pl.when(p > 0)
            def _():
                s_wait(rows1, osem1)

            g_start(c1, rows1, gsem1)
            g_wait(rows1, gsem1)
            s_start(c1, rows1, osem1)

            @pl.when(p < n_pairs - 1)
            def _():
                s_wait(rows0, osem0)
                g_start(c0 + 2, rows0, gsem0)

            return carry

        lax.fori_loop(0, n_pairs, body, 0)
        s_wait(rows0, osem0)
        s_wait(rows1, osem1)

    return gather_kernel


@jax.jit
def kernel(x, weight):
    B, H = x.shape
    V, D = weight.shape
    flat = x.reshape(B * H)
    out = _make_gather(V, D, B * H)(flat, weight)
    return out.reshape(B, H, D)


# speedup vs baseline: 1.7573x; 1.0000x over previous
import functools

import jax
import jax.numpy as jnp
from jax import lax
from jax.experimental import pallas as pl
from jax.experimental.layout import Format, Layout
from jax.experimental.pallas import tpu as pltpu
from jax.experimental.pallas import tpu_sc as plsc


@functools.cache
def _make_gather(V, D, B):
    info = plsc.get_sparse_core_info()
    NC, NS = info.num_cores, info.num_subcores
    NW = NC * NS
    assert B % NW == 0
    b_per_w = B // NW
    CHUNK = 800
    assert b_per_w % (2 * CHUNK) == 0
    n_pairs = b_per_w // (2 * CHUNK)
    mesh = plsc.VectorSubcoreMesh(core_axis_name="c", subcore_axis_name="s")

    @functools.partial(
        pl.kernel,
        mesh=mesh,
        out_type=jax.ShapeDtypeStruct((B, D), jnp.float32),
        scratch_types=[
            pltpu.VMEM((b_per_w,), jnp.int32),
            pltpu.VMEM((CHUNK, D), jnp.float32),
            pltpu.VMEM((CHUNK, D), jnp.float32),
            pltpu.SemaphoreType.DMA,
            pltpu.SemaphoreType.DMA,
            pltpu.SemaphoreType.DMA,
            pltpu.SemaphoreType.DMA,
        ],
        compiler_params=pltpu.CompilerParams(use_tc_tiling_on_sc=False),
    )
    def gather_kernel(idx_hbm, table_hbm, out_hbm, idx_all, rows0, rows1,
                      gsem0, gsem1, osem0, osem1):
        wid = lax.axis_index("s") * NC + lax.axis_index("c")
        base = wid * b_per_w
        pltpu.sync_copy(idx_hbm.at[pl.ds(base, b_per_w)], idx_all)

        def g_start(c, buf, sem):
            return pltpu.async_copy(
                table_hbm.at[idx_all.at[pl.ds(c * CHUNK, CHUNK)]], buf, sem)

        def s_start(c, buf, sem):
            return pltpu.async_copy(
                buf, out_hbm.at[pl.ds(base + c * CHUNK, CHUNK)], sem)

        def g_wait(buf, sem):
            pltpu.make_async_copy(table_hbm.at[idx_all.at[pl.ds(0, CHUNK)]],
                                  buf, sem).wait()

        def s_wait(buf, sem):
            pltpu.make_async_copy(buf, out_hbm.at[pl.ds(base, CHUNK)],
                                  sem).wait()

        g_start(0, rows0, gsem0)

        def body(p, carry):
            c0 = 2 * p
            c1 = c0 + 1
            g_wait(rows0, gsem0)
            s_start(c0, rows0, osem0)

            @pl.when(p > 0)
            def _():
                s_wait(rows1, osem1)

            g_start(c1, rows1, gsem1)
            g_wait(rows1, gsem1)
            s_start(c1, rows1, osem1)

            @pl.when(p < n_pairs - 1)
            def _():
                s_wait(rows0, osem0)
                g_start(c0 + 2, rows0, gsem0)

            return carry

        lax.fori_loop(0, n_pairs, body, 0)
        s_wait(rows0, osem0)
        s_wait(rows1, osem1)

    return gather_kernel


def _kernel_impl(x, weight):
    B, H = x.shape
    V, D = weight.shape
    flat = x.reshape(B * H)
    out = _make_gather(V, D, B * H)(flat, weight)
    return out.reshape(B, H, D)


@functools.cache
def _jitted_kernel():
    sharding = jax.sharding.SingleDeviceSharding(jax.devices()[0])
    fmt = Format(Layout(major_to_minor=(2, 1, 0)), sharding)
    return jax.jit(_kernel_impl, out_shardings=fmt)


def kernel(x, weight):
    return _jitted_kernel()(x, weight)
